# baseline (speedup 1.0000x reference)
import functools

import jax
import jax.numpy as jnp
from jax import lax
from jax.experimental import pallas as pl
from jax.experimental.pallas import tpu as pltpu
from jax.experimental.pallas import tpu_sc as plsc

_NC = 2
_NS = 16
_NW = _NC * _NS

_IDXW = 128
_K = 8
_CH = _K * _IDXW


def _sc_gather(x2d, table, n_tok, tok_dim):
    per_w = n_tok // _NW
    n_chunks = per_w // _CH

    mesh = plsc.VectorSubcoreMesh(core_axis_name="c", subcore_axis_name="s")

    @functools.partial(
        pl.kernel,
        mesh=mesh,
        compiler_params=pltpu.CompilerParams(use_tc_tiling_on_sc=False),
        out_type=jax.ShapeDtypeStruct((n_tok, tok_dim), jnp.float32),
        scratch_types=[
            pltpu.VMEM((_K, _IDXW), jnp.int32),
            pltpu.VMEM((_CH, tok_dim), jnp.float32),
            pltpu.SemaphoreType.DMA,
        ],
    )
    def gather_kernel(x_hbm, table_hbm, emb_hbm, idx_v, rows_v, sem):
        wid = lax.axis_index("s") * _NC + lax.axis_index("c")
        row0 = wid * (per_w // _IDXW)

        def body(c, carry):
            r = row0 + c * _K
            pltpu.sync_copy(x_hbm.at[pl.ds(r, _K)], idx_v)
            descs = [
                pltpu.async_copy(
                    table_hbm.at[idx_v.at[j]],
                    rows_v.at[pl.ds(j * _IDXW, _IDXW)],
                    sem,
                )
                for j in range(_K)
            ]
            for d in descs:
                d.wait()
            pltpu.sync_copy(rows_v, emb_hbm.at[pl.ds(r * _IDXW, _CH)])
            return carry

        lax.fori_loop(0, n_chunks, body, 0)

    return gather_kernel(x2d, table)


def _tc_project(xe, xo, emb2, wt, b2, n_half, tok_dim, emb_dim, bt):
    nb = n_half // bt

    def body(xe_ref, xo_ref, emb_ref, wt_ref, b2_ref, out_ref):
        me = (xe_ref[0, 0, :] != 0).astype(jnp.float32)[:, None]
        mo = (xo_ref[0, 0, :] != 0).astype(jnp.float32)[:, None]
        e = emb_ref[...]
        acc_e = jnp.dot(e[:, :tok_dim], wt_ref[...],
                        preferred_element_type=jnp.float32)
        acc_o = jnp.dot(e[:, tok_dim:], wt_ref[...],
                        preferred_element_type=jnp.float32)
        bias = jnp.concatenate(
            [me * b2_ref[:, :emb_dim], mo * b2_ref[:, emb_dim:]], axis=1)
        out_ref[...] = jnp.concatenate([acc_e, acc_o], axis=1) + bias

    return pl.pallas_call(
        body,
        grid=(nb,),
        in_specs=[
            pl.BlockSpec((1, 1, bt), lambda i: (i, 0, 0)),
            pl.BlockSpec((1, 1, bt), lambda i: (i, 0, 0)),
            pl.BlockSpec((bt, 2 * tok_dim), lambda i: (i, 0)),
            pl.BlockSpec((tok_dim, emb_dim), lambda i: (0, 0)),
            pl.BlockSpec((1, 2 * emb_dim), lambda i: (0, 0)),
        ],
        out_specs=pl.BlockSpec((bt, 2 * emb_dim), lambda i: (i, 0)),
        out_shape=jax.ShapeDtypeStruct((n_half, 2 * emb_dim), jnp.float32),
    )(xe, xo, emb2, wt, b2)


def kernel(x, table, W, b):
    bsz, seq = x.shape
    vocab, tok_dim = table.shape
    emb_dim = W.shape[0]
    n_tok = bsz * seq
    n_half = n_tok // 2

    xf = x.astype(jnp.int32).reshape(n_tok)
    x2d = xf.reshape(n_tok // _IDXW, _IDXW)

    emb = _sc_gather(x2d, table, n_tok, tok_dim)
    emb2 = emb.reshape(n_half, 2 * tok_dim)

    bt = 1024
    xe = xf[0::2].reshape(n_half // bt, 1, bt)
    xo = xf[1::2].reshape(n_half // bt, 1, bt)
    b2 = jnp.concatenate([b, b]).reshape(1, 2 * emb_dim)

    out2 = _tc_project(xe, xo, emb2, W.T, b2, n_half, tok_dim, emb_dim, bt)
    return out2.reshape(bsz, seq, emb_dim)


# speedup vs baseline: 1.1629x; 1.1629x over previous
import functools

import jax
import jax.numpy as jnp
from jax import lax
from jax.experimental import pallas as pl
from jax.experimental.pallas import tpu as pltpu
from jax.experimental.pallas import tpu_sc as plsc

_NC = 2
_NS = 16
_NW = _NC * _NS

_IDXW = 128
_K = 8
_CH = _K * _IDXW


def _sc_gather(xp2d, table, n_tok, tok_dim):
    per_w = n_tok // _NW
    n_chunks = per_w // _CH

    mesh = plsc.VectorSubcoreMesh(core_axis_name="c", subcore_axis_name="s")

    @functools.partial(
        pl.kernel,
        mesh=mesh,
        compiler_params=pltpu.CompilerParams(use_tc_tiling_on_sc=False),
        out_type=jax.ShapeDtypeStruct((n_tok, tok_dim), jnp.float32),
        scratch_types=[
            pltpu.VMEM((_K, _IDXW), jnp.int32),
            pltpu.VMEM((_CH, tok_dim), jnp.float32),
            pltpu.SemaphoreType.DMA,
        ],
    )
    def gather_kernel(x_hbm, table_hbm, emb_hbm, idx_v, rows_v, sem):
        wid = lax.axis_index("s") * _NC + lax.axis_index("c")
        row0 = wid * (per_w // _IDXW)

        def body(c, carry):
            r = row0 + c * _K
            pltpu.sync_copy(x_hbm.at[pl.ds(r, _K)], idx_v)
            descs = [
                pltpu.async_copy(
                    table_hbm.at[idx_v.at[j]],
                    rows_v.at[pl.ds(j * _IDXW, _IDXW)],
                    sem,
                )
                for j in range(_K)
            ]
            for d in descs:
                d.wait()
            pltpu.sync_copy(rows_v, emb_hbm.at[pl.ds(r * _IDXW, _CH)])
            return carry

        lax.fori_loop(0, n_chunks, body, 0)

    return gather_kernel(xp2d, table)


def _tc_project(xe, xo, emb2, w2, brow, n_half, emb_dim, bt):
    nb = n_half // bt

    def body(xe_ref, xo_ref, emb_ref, w2_ref, b_ref, out_ref):
        me = (xe_ref[0, 0, :] != 0).astype(jnp.float32)[:, None]
        mo = (xo_ref[0, 0, :] != 0).astype(jnp.float32)[:, None]
        acc = jnp.dot(emb_ref[...], w2_ref[...],
                      preferred_element_type=jnp.float32)
        out_ref[0] = acc[:, :emb_dim] + me * b_ref[...]
        out_ref[1] = acc[:, emb_dim:] + mo * b_ref[...]

    return pl.pallas_call(
        body,
        grid=(nb,),
        in_specs=[
            pl.BlockSpec((1, 1, bt), lambda i: (i, 0, 0)),
            pl.BlockSpec((1, 1, bt), lambda i: (i, 0, 0)),
            pl.BlockSpec((bt, w2.shape[0]), lambda i: (i, 0)),
            pl.BlockSpec(w2.shape, lambda i: (0, 0)),
            pl.BlockSpec((1, emb_dim), lambda i: (0, 0)),
        ],
        out_specs=pl.BlockSpec((2, bt, emb_dim), lambda i: (0, i, 0)),
        out_shape=jax.ShapeDtypeStruct((2, n_half, emb_dim), jnp.float32),
    )(xe, xo, emb2, w2, brow)


def kernel(x, table, W, b):
    bsz, seq = x.shape
    vocab, tok_dim = table.shape
    emb_dim = W.shape[0]
    n_tok = bsz * seq
    n_half = n_tok // 2

    xf = x.astype(jnp.int32).reshape(n_tok)
    x_lo, x_hi = xf[:n_half], xf[n_half:]
    xp2d = jnp.stack([x_lo, x_hi], axis=1).reshape(n_tok // _IDXW, _IDXW)

    emb2 = _sc_gather(xp2d, table, n_tok, tok_dim).reshape(n_half, 2 * tok_dim)

    wt = W.T
    w2 = jnp.zeros((2 * tok_dim, 2 * emb_dim), jnp.float32)
    w2 = w2.at[:tok_dim, :emb_dim].set(wt).at[tok_dim:, emb_dim:].set(wt)

    bt = 1024
    nb = n_half // bt
    out3 = _tc_project(
        x_lo.reshape(nb, 1, bt), x_hi.reshape(nb, 1, bt), emb2, w2,
        b.reshape(1, emb_dim), n_half, emb_dim, bt,
    )
    return out3.reshape(bsz, seq, emb_dim)


# speedup vs baseline: 1.5485x; 1.3316x over previous
import functools

import jax
import jax.numpy as jnp
from jax import lax
from jax.experimental import pallas as pl
from jax.experimental.pallas import tpu as pltpu
from jax.experimental.pallas import tpu_sc as plsc

_NC = 2
_NS = 16
_NW = _NC * _NS

_IDXW = 128
_K = 4
_CH = _K * _IDXW


def _tc_table_project(table, wt, brow, vocab, tok_dim, emb_dim, bv):
    nb = vocab // bv

    def body(t_ref, wt_ref, b_ref, out_ref):
        acc = jnp.dot(t_ref[...], wt_ref[...],
                      preferred_element_type=jnp.float32)
        out_ref[...] = acc + b_ref[...]

        @pl.when(pl.program_id(0) == 0)
        def _():
            out_ref[0:1, :] = jnp.zeros((1, emb_dim), jnp.float32)

    return pl.pallas_call(
        body,
        grid=(nb,),
        in_specs=[
            pl.BlockSpec((bv, tok_dim), lambda i: (i, 0)),
            pl.BlockSpec((tok_dim, emb_dim), lambda i: (0, 0)),
            pl.BlockSpec((1, emb_dim), lambda i: (0, 0)),
        ],
        out_specs=pl.BlockSpec((bv, emb_dim), lambda i: (i, 0)),
        out_shape=jax.ShapeDtypeStruct((vocab, emb_dim), jnp.float32),
    )(table, wt, brow)


def _sc_gather(x2d, tw, n_tok, emb_dim):
    per_w = n_tok // _NW
    n_chunks = per_w // _CH

    mesh = plsc.VectorSubcoreMesh(core_axis_name="c", subcore_axis_name="s")

    @functools.partial(
        pl.kernel,
        mesh=mesh,
        compiler_params=pltpu.CompilerParams(use_tc_tiling_on_sc=False),
        out_type=jax.ShapeDtypeStruct((n_tok, emb_dim), jnp.float32),
        scratch_types=[
            pltpu.VMEM((_K, _IDXW), jnp.int32),
            pltpu.VMEM((_CH, emb_dim), jnp.float32),
            pltpu.SemaphoreType.DMA,
        ],
    )
    def gather_kernel(x_hbm, tw_hbm, out_hbm, idx_v, rows_v, sem):
        wid = lax.axis_index("s") * _NC + lax.axis_index("c")
        row0 = wid * (per_w // _IDXW)

        def body(c, carry):
            r = row0 + c * _K
            pltpu.sync_copy(x_hbm.at[pl.ds(r, _K)], idx_v)
            descs = [
                pltpu.async_copy(
                    tw_hbm.at[idx_v.at[j]],
                    rows_v.at[pl.ds(j * _IDXW, _IDXW)],
                    sem,
                )
                for j in range(_K)
            ]
            for d in descs:
                d.wait()
            pltpu.sync_copy(rows_v, out_hbm.at[pl.ds(r * _IDXW, _CH)])
            return carry

        lax.fori_loop(0, n_chunks, body, 0)

    return gather_kernel(x2d, tw)


def kernel(x, table, W, b):
    bsz, seq = x.shape
    vocab, tok_dim = table.shape
    emb_dim = W.shape[0]
    n_tok = bsz * seq

    tw = _tc_table_project(
        table, W.T, b.reshape(1, emb_dim), vocab, tok_dim, emb_dim, bv=8000)

    x2d = x.astype(jnp.int32).reshape(n_tok // _IDXW, _IDXW)
    out = _sc_gather(x2d, tw, n_tok, emb_dim)
    return out.reshape(bsz, seq, emb_dim)


# speedup vs baseline: 2.4993x; 1.6140x over previous
import functools

import jax
import jax.numpy as jnp
from jax import lax
from jax.experimental import pallas as pl
from jax.experimental.pallas import tpu as pltpu
from jax.experimental.pallas import tpu_sc as plsc

_NC = 2
_NS = 16
_NW = _NC * _NS

_IDXW = 128
_K = 4
_CH = _K * _IDXW


def _tc_table_project(table, wt, brow, vocab, tok_dim, emb_dim, bv):
    nb = pl.cdiv(vocab, bv)

    def body(t_ref, wt_ref, b_ref, out_ref):
        acc = lax.dot_general(
            t_ref[...], wt_ref[...],
            dimension_numbers=(((0,), (0,)), ((), ())),
            preferred_element_type=jnp.float32)
        out_ref[...] = acc + b_ref[...]

        @pl.when(pl.program_id(0) == 0)
        def _():
            out_ref[0:1, :] = jnp.zeros((1, emb_dim), jnp.float32)

    return pl.pallas_call(
        body,
        grid=(nb,),
        in_specs=[
            pl.BlockSpec((tok_dim, bv), lambda i: (0, i)),
            pl.BlockSpec((tok_dim, emb_dim), lambda i: (0, 0)),
            pl.BlockSpec((1, emb_dim), lambda i: (0, 0)),
        ],
        out_specs=pl.BlockSpec((bv, emb_dim), lambda i: (i, 0)),
        out_shape=jax.ShapeDtypeStruct((vocab, emb_dim), jnp.float32),
    )(table.T, wt, brow)


def _sc_gather(x2d, tw, n_tok, emb_dim):
    per_w = n_tok // _NW
    n_chunks = per_w // _CH

    mesh = plsc.VectorSubcoreMesh(core_axis_name="c", subcore_axis_name="s")

    @functools.partial(
        pl.kernel,
        mesh=mesh,
        compiler_params=pltpu.CompilerParams(use_tc_tiling_on_sc=False),
        out_type=jax.ShapeDtypeStruct((n_tok, emb_dim), jnp.float32),
        scratch_types=[
            pltpu.VMEM((_K, _IDXW), jnp.int32),
            pltpu.VMEM((_CH, emb_dim), jnp.float32),
            pltpu.SemaphoreType.DMA,
        ],
    )
    def gather_kernel(x_hbm, tw_hbm, out_hbm, idx_v, rows_v, sem):
        wid = lax.axis_index("s") * _NC + lax.axis_index("c")
        row0 = wid * (per_w // _IDXW)

        def body(c, carry):
            r = row0 + c * _K
            pltpu.sync_copy(x_hbm.at[pl.ds(r, _K)], idx_v)
            descs = [
                pltpu.async_copy(
                    tw_hbm.at[idx_v.at[j]],
                    rows_v.at[pl.ds(j * _IDXW, _IDXW)],
                    sem,
                )
                for j in range(_K)
            ]
            for d in descs:
                d.wait()
            pltpu.sync_copy(rows_v, out_hbm.at[pl.ds(r * _IDXW, _CH)])
            return carry

        lax.fori_loop(0, n_chunks, body, 0)

    return gather_kernel(x2d, tw)


def kernel(x, table, W, b):
    bsz, seq = x.shape
    vocab, tok_dim = table.shape
    emb_dim = W.shape[0]
    n_tok = bsz * seq

    tw = _tc_table_project(
        table, W.T, b.reshape(1, emb_dim), vocab, tok_dim, emb_dim, bv=8192)

    x2d = x.astype(jnp.int32).reshape(n_tok // _IDXW, _IDXW)
    out = _sc_gather(x2d, tw, n_tok, emb_dim)
    return out.reshape(bsz, seq, emb_dim)


# speedup vs baseline: 2.5705x; 1.0285x over previous
import functools

import jax
import jax.numpy as jnp
from jax import lax
from jax.experimental import pallas as pl
from jax.experimental.pallas import tpu as pltpu
from jax.experimental.pallas import tpu_sc as plsc

_NC = 2
_NS = 16
_NW = _NC * _NS

_IDXW = 128
_K = 4
_CH = _K * _IDXW


def _tc_table_project(table, wt, brow, vocab, tok_dim, emb_dim, bv):
    nb = pl.cdiv(vocab, bv)

    def body(t_ref, wt_ref, b_ref, out_ref):
        acc = lax.dot_general(
            t_ref[...].astype(jnp.bfloat16), wt_ref[...].astype(jnp.bfloat16),
            dimension_numbers=(((0,), (0,)), ((), ())),
            preferred_element_type=jnp.float32)
        out_ref[...] = acc + b_ref[...]

        @pl.when(pl.program_id(0) == 0)
        def _():
            out_ref[0:1, :] = jnp.zeros((1, emb_dim), jnp.float32)

    return pl.pallas_call(
        body,
        grid=(nb,),
        in_specs=[
            pl.BlockSpec((tok_dim, bv), lambda i: (0, i)),
            pl.BlockSpec((tok_dim, emb_dim), lambda i: (0, 0)),
            pl.BlockSpec((1, emb_dim), lambda i: (0, 0)),
        ],
        out_specs=pl.BlockSpec((bv, emb_dim), lambda i: (i, 0)),
        out_shape=jax.ShapeDtypeStruct((vocab, emb_dim), jnp.float32),
    )(table.T, wt, brow)


def _sc_gather(x2d, tw, n_tok, emb_dim):
    per_w = n_tok // _NW
    n_chunks = per_w // _CH

    mesh = plsc.VectorSubcoreMesh(core_axis_name="c", subcore_axis_name="s")

    @functools.partial(
        pl.kernel,
        mesh=mesh,
        compiler_params=pltpu.CompilerParams(use_tc_tiling_on_sc=False),
        out_type=jax.ShapeDtypeStruct((n_tok, emb_dim), jnp.float32),
        scratch_types=[
            pltpu.VMEM((_K, _IDXW), jnp.int32),
            pltpu.VMEM((_CH, emb_dim), jnp.float32),
            pltpu.SemaphoreType.DMA,
        ],
    )
    def gather_kernel(x_hbm, tw_hbm, out_hbm, idx_v, rows_v, sem):
        wid = lax.axis_index("s") * _NC + lax.axis_index("c")
        row0 = wid * (per_w // _IDXW)

        def body(c, carry):
            r = row0 + c * _K
            pltpu.sync_copy(x_hbm.at[pl.ds(r, _K)], idx_v)
            descs = [
                pltpu.async_copy(
                    tw_hbm.at[idx_v.at[j]],
                    rows_v.at[pl.ds(j * _IDXW, _IDXW)],
                    sem,
                )
                for j in range(_K)
            ]
            for d in descs:
                d.wait()
            pltpu.sync_copy(rows_v, out_hbm.at[pl.ds(r * _IDXW, _CH)])
            return carry

        lax.fori_loop(0, n_chunks, body, 0)

    return gather_kernel(x2d, tw)


def kernel(x, table, W, b):
    bsz, seq = x.shape
    vocab, tok_dim = table.shape
    emb_dim = W.shape[0]
    n_tok = bsz * seq

    tw = _tc_table_project(
        table, W.T, b.reshape(1, emb_dim), vocab, tok_dim, emb_dim, bv=8192)

    x2d = x.astype(jnp.int32).reshape(n_tok // _IDXW, _IDXW)
    out = _sc_gather(x2d, tw, n_tok, emb_dim)
    return out.reshape(bsz, seq, emb_dim)
